# baseline (speedup 1.0000x reference)
import functools

import jax
import jax.numpy as jnp
from jax import lax
from jax.experimental import pallas as pl
from jax.experimental.pallas import tpu as pltpu
from jax.experimental.pallas import tpu_sc as plsc

N = 10000
E = 320000
D = 128
NG = 16

NC = 2
NS = 16
NW = NC * NS

NPAD = 10240
ROWS_PT = NPAD // NS

B = 64
NBUF = 4
NCHUNK = 160
EPW = NCHUNK * B
EPAD = EPW * NW
NGROUP = NCHUNK // NBUF
CW = 128
BD = 128
NCHUNK_D = EPW // BD
DEG_WIN = 8

_mesh = plsc.VectorSubcoreMesh(core_axis_name="c", subcore_axis_name="s")


@functools.partial(
    pl.kernel,
    out_type=jax.ShapeDtypeStruct((NC, NPAD, CW), jnp.float32),
    mesh=_mesh,
    scratch_types=[
        pltpu.VMEM((NCHUNK_D, BD), jnp.int32),
        pltpu.VMEM((BD, CW), jnp.float32),
        pltpu.VMEM_SHARED((NPAD, CW), jnp.float32),
        pltpu.SemaphoreType.DMA,
    ],
)
def _deg_kernel(col_hbm, ones_hbm, zeros_hbm, out_hbm, coli_v, ones_v, cnt_sh, sem):
    c = lax.axis_index("c")
    s = lax.axis_index("s")
    wid = c * NS + s
    r0 = s * ROWS_PT
    pltpu.sync_copy(zeros_hbm.at[pl.ds(r0, ROWS_PT)], cnt_sh.at[pl.ds(r0, ROWS_PT)])
    pltpu.sync_copy(col_hbm.at[wid], coli_v)
    pltpu.sync_copy(ones_hbm, ones_v)
    plsc.subcore_barrier()

    def fire(j):
        pltpu.async_copy(ones_v, cnt_sh.at[coli_v.at[j]], sem, add=True)

    def drain_one():
        pltpu.make_async_copy(ones_v, cnt_sh.at[coli_v.at[0]], sem).wait()

    for j in range(DEG_WIN):
        fire(j)

    def body(j, carry):
        drain_one()
        fire(j)
        return carry

    lax.fori_loop(DEG_WIN, NCHUNK_D, body, 0)
    for _ in range(DEG_WIN):
        drain_one()
    plsc.subcore_barrier()
    pltpu.sync_copy(cnt_sh.at[pl.ds(r0, ROWS_PT)], out_hbm.at[c, pl.ds(r0, ROWS_PT)])


@functools.partial(
    pl.kernel,
    out_type=jax.ShapeDtypeStruct((NC, NPAD, D), jnp.float32),
    mesh=_mesh,
    scratch_types=[
        pltpu.VMEM((2, NBUF, B), jnp.int32),
        pltpu.VMEM((2, NBUF, B), jnp.int32),
        pltpu.VMEM((NBUF, B, D), jnp.float32),
        pltpu.VMEM_SHARED((NPAD, D), jnp.float32),
        pltpu.SemaphoreType.DMA((NBUF,)),
        pltpu.SemaphoreType.DMA((NBUF,)),
        pltpu.SemaphoreType.DMA,
    ],
)
def _agg_kernel(y_hbm, row_hbm, col_hbm, zeros_hbm, out_hbm,
                rowi_v, coli_v, bufs_v, acc_sh, gsem, ssem, isem):
    c = lax.axis_index("c")
    s = lax.axis_index("s")
    wid = c * NS + s
    r0 = s * ROWS_PT

    def idx_load(g_src, p, sync=False):
        if sync:
            pltpu.sync_copy(row_hbm.at[wid, g_src], rowi_v.at[p])
            pltpu.sync_copy(col_hbm.at[wid, g_src], coli_v.at[p])
        else:
            pltpu.async_copy(row_hbm.at[wid, g_src], rowi_v.at[p], isem)
            pltpu.async_copy(col_hbm.at[wid, g_src], coli_v.at[p], isem)

    def idx_wait():
        pltpu.make_async_copy(row_hbm.at[wid, 0], rowi_v.at[0], isem).wait()
        pltpu.make_async_copy(col_hbm.at[wid, 0], coli_v.at[0], isem).wait()

    def gather_start(p, b):
        pltpu.async_copy(y_hbm.at[rowi_v.at[p, b]], bufs_v.at[b], gsem.at[b])

    def gather_wait(b):
        pltpu.make_async_copy(y_hbm.at[rowi_v.at[0, 0]], bufs_v.at[b],
                              gsem.at[b]).wait()

    def scatter_start(p, b):
        pltpu.async_copy(bufs_v.at[b], acc_sh.at[coli_v.at[p, b]], ssem.at[b],
                         add=True)

    def scatter_wait(b):
        pltpu.make_async_copy(bufs_v.at[b], acc_sh.at[coli_v.at[0, 0]],
                              ssem.at[b]).wait()

    idx_load(0, 0, sync=True)
    for b in range(NBUF):
        gather_start(0, b)
    idx_load(1, 1)
    pltpu.sync_copy(zeros_hbm.at[pl.ds(r0, ROWS_PT)], acc_sh.at[pl.ds(r0, ROWS_PT)])
    plsc.subcore_barrier()

    def group(g, carry):
        p = lax.rem(g, 2)
        for b in range(NBUF):
            gather_wait(b)
            scatter_start(p, b)
        idx_wait()
        for b in range(NBUF):
            scatter_wait(b)
            gather_start(1 - p, b)
        idx_load(jnp.minimum(g + 2, NGROUP - 1), p)
        return carry

    lax.fori_loop(0, NGROUP - 1, group, 0)
    plast = (NGROUP - 1) % 2
    for b in range(NBUF):
        gather_wait(b)
        scatter_start(plast, b)
    idx_wait()
    for b in range(NBUF):
        scatter_wait(b)
    plsc.subcore_barrier()
    pltpu.sync_copy(acc_sh.at[pl.ds(r0, ROWS_PT)], out_hbm.at[c, pl.ds(r0, ROWS_PT)])


GRID = 8
RB = NPAD // GRID


def _dis_from_cnt(cnt_t_ref):
    deg = cnt_t_ref[:, 0:1] + cnt_t_ref[:, 1:2] + 1.0
    return lax.rsqrt(deg)


def _row_spec(minor=D):
    return pl.BlockSpec((RB, minor), lambda i: (i, 0))


def _full_spec(shape):
    nd = len(shape)
    return pl.BlockSpec(shape, lambda i, _n=nd: (0,) * _n)


def _tc_xw_body(x_ref, w1_ref, xw_ref):
    xw_ref[...] = jnp.dot(x_ref[...], w1_ref[...],
                          preferred_element_type=jnp.float32)


_tc_xw = pl.pallas_call(
    _tc_xw_body,
    grid=(GRID,),
    in_specs=[_row_spec(), _full_spec((D, D))],
    out_specs=_row_spec(),
    out_shape=jax.ShapeDtypeStruct((NPAD, D), jnp.float32),
)


def _tc_scale_body(cnt_t_ref, xw_ref, y_ref):
    y_ref[...] = xw_ref[...] * _dis_from_cnt(cnt_t_ref)


_tc_scale = pl.pallas_call(
    _tc_scale_body,
    grid=(GRID,),
    in_specs=[_row_spec(2), _row_spec()],
    out_specs=_row_spec(),
    out_shape=jax.ShapeDtypeStruct((NPAD, D), jnp.float32),
)


def _tc_mid_body(acc_ref, cnt_t_ref, xw_ref, b1_ref, w2_ref, y2_ref, xw2_ref):
    dis = _dis_from_cnt(cnt_t_ref)
    agg = acc_ref[0] + acc_ref[1]
    h = jax.nn.relu(dis * agg + (dis * dis) * xw_ref[...] + b1_ref[...])
    xw2 = jnp.dot(h, w2_ref[...], preferred_element_type=jnp.float32)
    xw2_ref[...] = xw2
    y2_ref[...] = xw2 * dis


_tc_mid = pl.pallas_call(
    _tc_mid_body,
    grid=(GRID,),
    in_specs=[
        pl.BlockSpec((NC, RB, D), lambda i: (0, i, 0)),
        _row_spec(2),
        _row_spec(),
        _full_spec((1, D)),
        _full_spec((D, D)),
    ],
    out_specs=[_row_spec(), _row_spec()],
    out_shape=[
        jax.ShapeDtypeStruct((NPAD, D), jnp.float32),
        jax.ShapeDtypeStruct((NPAD, D), jnp.float32),
    ],
)


def _tc_fin_body(acc_ref, cnt_t_ref, xw_ref, b2_ref, batch_ref,
                 wl1_ref, bl1_ref, wl2_ref, bl2_ref, out_ref,
                 sums_scr, cnts_scr):
    i = pl.program_id(0)

    @pl.when(i == 0)
    def _init():
        sums_scr[...] = jnp.zeros((NG, D), jnp.float32)
        cnts_scr[...] = jnp.zeros((NG, CW), jnp.float32)

    dis = _dis_from_cnt(cnt_t_ref)
    agg = acc_ref[0] + acc_ref[1]
    h = jax.nn.relu(dis * agg + (dis * dis) * xw_ref[...] + b2_ref[...])
    gidx = lax.broadcasted_iota(jnp.int32, (NG, RB), 0)
    mask = (jnp.broadcast_to(batch_ref[...], (NG, RB)) == gidx).astype(jnp.float32)
    sums_scr[...] += jnp.dot(mask, h, preferred_element_type=jnp.float32)
    cnts_scr[...] += jnp.broadcast_to(
        jnp.sum(mask, axis=1, keepdims=True), (NG, CW))

    @pl.when(i == GRID - 1)
    def _head():
        cnt = jnp.maximum(cnts_scr[:, 0:1], 1.0)
        g = sums_scr[...] / cnt
        g = jax.nn.relu(jnp.dot(g, wl1_ref[...],
                                preferred_element_type=jnp.float32)
                        + bl1_ref[...])
        out_ref[...] = (jnp.dot(g, wl2_ref[...],
                                preferred_element_type=jnp.float32)
                        + bl2_ref[...])


_tc_fin = pl.pallas_call(
    _tc_fin_body,
    grid=(GRID,),
    in_specs=[
        pl.BlockSpec((NC, RB, D), lambda i: (0, i, 0)),
        _row_spec(2),
        _row_spec(),
        _full_spec((1, D)),
        pl.BlockSpec((1, RB), lambda i: (0, i)),
        _full_spec((D, D)),
        _full_spec((1, D)),
        _full_spec((D, 16)),
        _full_spec((1, 16)),
    ],
    out_specs=_full_spec((NG, 16)),
    out_shape=jax.ShapeDtypeStruct((NG, 16), jnp.float32),
    scratch_shapes=[
        pltpu.VMEM((NG, D), jnp.float32),
        pltpu.VMEM((NG, CW), jnp.float32),
    ],
)


def kernel(x, edge_index, batch, W1, b1, W2, b2, Wl1, bl1, Wl2, bl2):
    row = edge_index[0].astype(jnp.int32)
    col = edge_index[1].astype(jnp.int32)
    npadrows = NPAD - N
    pad_col = (N + jnp.arange(EPAD - E, dtype=jnp.int32) % npadrows)
    row_p = jnp.pad(row, (0, EPAD - E), constant_values=0)
    col_p = jnp.concatenate([col, pad_col])
    row_g = row_p.reshape(NW, NGROUP, NBUF, B)
    col_g = col_p.reshape(NW, NGROUP, NBUF, B)
    col_c = col_p.reshape(NW, NCHUNK_D, BD)

    x_p = jnp.pad(x, ((0, NPAD - N), (0, 0)))
    batch_p = jnp.pad(batch.astype(jnp.int32), (0, NPAD - N),
                      constant_values=NG).reshape(1, NPAD)

    zeros2d = jnp.zeros((NPAD, D), jnp.float32)
    ones_c = jnp.ones((BD, CW), jnp.float32)

    cnt = _deg_kernel(col_c, ones_c, zeros2d)
    cnt_t = cnt[:, :, 0].T

    xw1 = _tc_xw(x_p, W1)
    y1 = _tc_scale(cnt_t, xw1)
    acc1 = _agg_kernel(y1, row_g, col_g, zeros2d)
    y2, xw2 = _tc_mid(acc1, cnt_t, xw1, b1.reshape(1, D), W2)
    acc2 = _agg_kernel(y2, row_g, col_g, zeros2d)
    out = _tc_fin(acc2, cnt_t, xw2, b2.reshape(1, D), batch_p,
                  Wl1, bl1.reshape(1, D), Wl2, bl2.reshape(1, 16))
    return out


# speedup vs baseline: 2.4504x; 2.4504x over previous
import functools

import jax
import jax.numpy as jnp
from jax import lax
from jax.experimental import pallas as pl
from jax.experimental.pallas import tpu as pltpu
from jax.experimental.pallas import tpu_sc as plsc

N = 10000
E = 320000
D = 128
NG = 16

NC = 2
NS = 16
NW = NC * NS

NPAD = 10240
ROWS_PT = NPAD // NS

B = 64
NBUF = 4
NCHUNK = 160
EPW = NCHUNK * B
EPAD = EPW * NW
NGROUP = NCHUNK // NBUF
CW = 128
BD = 128
NCHUNK_D = EPW // BD
DEG_WIN = 8

_mesh = plsc.VectorSubcoreMesh(core_axis_name="c", subcore_axis_name="s")


@functools.partial(
    pl.kernel,
    out_type=jax.ShapeDtypeStruct((NC, NPAD, CW), jnp.float32),
    mesh=_mesh,
    scratch_types=[
        pltpu.VMEM((NCHUNK_D, BD), jnp.int32),
        pltpu.VMEM((BD, CW), jnp.float32),
        pltpu.VMEM_SHARED((NPAD, CW), jnp.float32),
        pltpu.SemaphoreType.DMA,
    ],
)
def _deg_kernel(col_hbm, ones_hbm, zeros_hbm, out_hbm, coli_v, ones_v, cnt_sh, sem):
    c = lax.axis_index("c")
    s = lax.axis_index("s")
    wid = c * NS + s
    r0 = s * ROWS_PT
    pltpu.sync_copy(zeros_hbm.at[pl.ds(r0, ROWS_PT)], cnt_sh.at[pl.ds(r0, ROWS_PT)])
    pltpu.sync_copy(col_hbm.at[wid], coli_v)
    pltpu.sync_copy(ones_hbm, ones_v)
    plsc.subcore_barrier()

    def fire(j):
        pltpu.async_copy(ones_v, cnt_sh.at[coli_v.at[j]], sem, add=True)

    def drain_one():
        pltpu.make_async_copy(ones_v, cnt_sh.at[coli_v.at[0]], sem).wait()

    for j in range(DEG_WIN):
        fire(j)

    def body(j, carry):
        drain_one()
        fire(j)
        return carry

    lax.fori_loop(DEG_WIN, NCHUNK_D, body, 0)
    for _ in range(DEG_WIN):
        drain_one()
    plsc.subcore_barrier()
    pltpu.sync_copy(cnt_sh.at[pl.ds(r0, ROWS_PT)], out_hbm.at[c, pl.ds(r0, ROWS_PT)])


@functools.partial(
    pl.kernel,
    out_type=jax.ShapeDtypeStruct((NC, NPAD, D), jnp.float32),
    mesh=_mesh,
    scratch_types=[
        pltpu.VMEM((2, NBUF, B), jnp.int32),
        pltpu.VMEM((2, NBUF, B), jnp.int32),
        pltpu.VMEM((NBUF, B, D), jnp.float32),
        pltpu.VMEM_SHARED((NPAD, D), jnp.float32),
        pltpu.SemaphoreType.DMA((NBUF,)),
        pltpu.SemaphoreType.DMA((NBUF,)),
        pltpu.SemaphoreType.DMA,
    ],
)
def _agg_kernel(y_hbm, row_hbm, col_hbm, zeros_hbm, out_hbm,
                rowi_v, coli_v, bufs_v, acc_sh, gsem, ssem, isem):
    c = lax.axis_index("c")
    s = lax.axis_index("s")
    wid = c * NS + s
    r0 = s * ROWS_PT

    def idx_load(g_src, p, sync=False):
        if sync:
            pltpu.sync_copy(row_hbm.at[wid, g_src], rowi_v.at[p])
            pltpu.sync_copy(col_hbm.at[wid, g_src], coli_v.at[p])
        else:
            pltpu.async_copy(row_hbm.at[wid, g_src], rowi_v.at[p], isem)
            pltpu.async_copy(col_hbm.at[wid, g_src], coli_v.at[p], isem)

    def idx_wait():
        pltpu.make_async_copy(row_hbm.at[wid, 0], rowi_v.at[0], isem).wait()
        pltpu.make_async_copy(col_hbm.at[wid, 0], coli_v.at[0], isem).wait()

    def gather_start(p, b):
        pltpu.async_copy(y_hbm.at[rowi_v.at[p, b]], bufs_v.at[b], gsem.at[b])

    def gather_wait(b):
        pltpu.make_async_copy(y_hbm.at[rowi_v.at[0, 0]], bufs_v.at[b],
                              gsem.at[b]).wait()

    def scatter_start(p, b):
        pltpu.async_copy(bufs_v.at[b], acc_sh.at[coli_v.at[p, b]], ssem.at[b],
                         add=True)

    def scatter_wait(b):
        pltpu.make_async_copy(bufs_v.at[b], acc_sh.at[coli_v.at[0, 0]],
                              ssem.at[b]).wait()

    idx_load(0, 0, sync=True)
    for b in range(NBUF):
        gather_start(0, b)
    idx_load(1, 1)
    pltpu.sync_copy(zeros_hbm.at[pl.ds(r0, ROWS_PT)], acc_sh.at[pl.ds(r0, ROWS_PT)])
    plsc.subcore_barrier()

    def group(g, carry):
        p = lax.rem(g, 2)
        for b in range(NBUF):
            gather_wait(b)
            scatter_start(p, b)
        idx_wait()
        for b in range(NBUF):
            scatter_wait(b)
            gather_start(1 - p, b)
        idx_load(jnp.minimum(g + 2, NGROUP - 1), p)
        return carry

    lax.fori_loop(0, NGROUP - 1, group, 0)
    plast = (NGROUP - 1) % 2
    for b in range(NBUF):
        gather_wait(b)
        scatter_start(plast, b)
    idx_wait()
    for b in range(NBUF):
        scatter_wait(b)
    plsc.subcore_barrier()
    pltpu.sync_copy(acc_sh.at[pl.ds(r0, ROWS_PT)], out_hbm.at[c, pl.ds(r0, ROWS_PT)])


GRID = 8
RB = NPAD // GRID


def _dis_from_cnt(cnt_t_ref):
    deg = cnt_t_ref[:, 0:1] + cnt_t_ref[:, 1:2] + 1.0
    return lax.rsqrt(deg)


def _row_spec(minor=D):
    return pl.BlockSpec((RB, minor), lambda i: (i, 0))


def _full_spec(shape):
    nd = len(shape)
    return pl.BlockSpec(shape, lambda i, _n=nd: (0,) * _n)


def _tc_xw_body(x_ref, w1_ref, xw_ref):
    xw_ref[...] = jnp.dot(x_ref[...], w1_ref[...],
                          preferred_element_type=jnp.float32)


_tc_xw = pl.pallas_call(
    _tc_xw_body,
    grid=(GRID,),
    in_specs=[_row_spec(), _full_spec((D, D))],
    out_specs=_row_spec(),
    out_shape=jax.ShapeDtypeStruct((NPAD, D), jnp.float32),
)


def _tc_scale_body(cnt_t_ref, xw_ref, y_ref):
    y_ref[...] = xw_ref[...] * _dis_from_cnt(cnt_t_ref)


_tc_scale = pl.pallas_call(
    _tc_scale_body,
    grid=(GRID,),
    in_specs=[_row_spec(2), _row_spec()],
    out_specs=_row_spec(),
    out_shape=jax.ShapeDtypeStruct((NPAD, D), jnp.float32),
)


def _tc_mid_body(acc_ref, cnt_t_ref, xw_ref, b1_ref, w2_ref, y2_ref, xw2_ref):
    dis = _dis_from_cnt(cnt_t_ref)
    agg = acc_ref[0] + acc_ref[1]
    h = jax.nn.relu(dis * agg + (dis * dis) * xw_ref[...] + b1_ref[...])
    xw2 = jnp.dot(h, w2_ref[...], preferred_element_type=jnp.float32)
    xw2_ref[...] = xw2
    y2_ref[...] = xw2 * dis


_tc_mid = pl.pallas_call(
    _tc_mid_body,
    grid=(GRID,),
    in_specs=[
        pl.BlockSpec((NC, RB, D), lambda i: (0, i, 0)),
        _row_spec(2),
        _row_spec(),
        _full_spec((1, D)),
        _full_spec((D, D)),
    ],
    out_specs=[_row_spec(), _row_spec()],
    out_shape=[
        jax.ShapeDtypeStruct((NPAD, D), jnp.float32),
        jax.ShapeDtypeStruct((NPAD, D), jnp.float32),
    ],
)


def _tc_fin_body(acc_ref, cnt_t_ref, xw_ref, b2_ref, batch_ref,
                 wl1_ref, bl1_ref, wl2_ref, bl2_ref, out_ref,
                 sums_scr, cnts_scr):
    i = pl.program_id(0)

    @pl.when(i == 0)
    def _init():
        sums_scr[...] = jnp.zeros((NG, D), jnp.float32)
        cnts_scr[...] = jnp.zeros((NG, CW), jnp.float32)

    dis = _dis_from_cnt(cnt_t_ref)
    agg = acc_ref[0] + acc_ref[1]
    h = jax.nn.relu(dis * agg + (dis * dis) * xw_ref[...] + b2_ref[...])
    gidx = lax.broadcasted_iota(jnp.int32, (NG, RB), 0)
    mask = (jnp.broadcast_to(batch_ref[...], (NG, RB)) == gidx).astype(jnp.float32)
    sums_scr[...] += jnp.dot(mask, h, preferred_element_type=jnp.float32)
    cnts_scr[...] += jnp.broadcast_to(
        jnp.sum(mask, axis=1, keepdims=True), (NG, CW))

    @pl.when(i == GRID - 1)
    def _head():
        cnt = jnp.maximum(cnts_scr[:, 0:1], 1.0)
        g = sums_scr[...] / cnt
        g = jax.nn.relu(jnp.dot(g, wl1_ref[...],
                                preferred_element_type=jnp.float32)
                        + bl1_ref[...])
        out_ref[...] = (jnp.dot(g, wl2_ref[...],
                                preferred_element_type=jnp.float32)
                        + bl2_ref[...])


_tc_fin = pl.pallas_call(
    _tc_fin_body,
    grid=(GRID,),
    in_specs=[
        pl.BlockSpec((NC, RB, D), lambda i: (0, i, 0)),
        _row_spec(2),
        _row_spec(),
        _full_spec((1, D)),
        pl.BlockSpec((1, RB), lambda i: (0, i)),
        _full_spec((D, D)),
        _full_spec((1, D)),
        _full_spec((D, 16)),
        _full_spec((1, 16)),
    ],
    out_specs=_full_spec((NG, 16)),
    out_shape=jax.ShapeDtypeStruct((NG, 16), jnp.float32),
    scratch_shapes=[
        pltpu.VMEM((NG, D), jnp.float32),
        pltpu.VMEM((NG, CW), jnp.float32),
    ],
)


def kernel(x, edge_index, batch, W1, b1, W2, b2, Wl1, bl1, Wl2, bl2):
    row = edge_index[0].astype(jnp.int32)
    col = edge_index[1].astype(jnp.int32)
    npadrows = NPAD - N
    kpad = jnp.arange(EPAD - E, dtype=jnp.int32)
    pad_col = N + kpad % npadrows
    pad_row = kpad % N
    row_p = jnp.concatenate([row, pad_row])
    col_p = jnp.concatenate([col, pad_col])
    row_w = row_p.reshape(EPW, NW).T
    col_w = col_p.reshape(EPW, NW).T
    row_g = row_w.reshape(NW, NGROUP, NBUF, B)
    col_g = col_w.reshape(NW, NGROUP, NBUF, B)
    col_c = col_w.reshape(NW, NCHUNK_D, BD)

    x_p = jnp.pad(x, ((0, NPAD - N), (0, 0)))
    batch_p = jnp.pad(batch.astype(jnp.int32), (0, NPAD - N),
                      constant_values=NG).reshape(1, NPAD)

    zeros2d = jnp.zeros((NPAD, D), jnp.float32)
    ones_c = jnp.ones((BD, CW), jnp.float32)

    cnt = _deg_kernel(col_c, ones_c, zeros2d)
    cnt_t = cnt[:, :, 0].T

    xw1 = _tc_xw(x_p, W1)
    y1 = _tc_scale(cnt_t, xw1)
    acc1 = _agg_kernel(y1, row_g, col_g, zeros2d)
    y2, xw2 = _tc_mid(acc1, cnt_t, xw1, b1.reshape(1, D), W2)
    acc2 = _agg_kernel(y2, row_g, col_g, zeros2d)
    out = _tc_fin(acc2, cnt_t, xw2, b2.reshape(1, D), batch_p,
                  Wl1, bl1.reshape(1, D), Wl2, bl2.reshape(1, 16))
    return out
